# baseline (speedup 1.0000x reference)
import dataclasses

import jax
import jax.numpy as jnp
from jax.experimental import pallas as pl
from jax.experimental.pallas import tpu as pltpu
from jax.experimental.pallas import tpu_sc as plsc

_LANES = 16
_CHROWS = 32
_NBUF = 4
_PD = 2


def _sc_body(gu_hbm, gi_hbm, xui_hbm, ub, vb, xs, xacc, su, sv, sx):
    B, D = gu_hbm.shape
    n_tecs = 32
    rows_per_tec = B // n_tecs
    nc = rows_per_tec // _CHROWS
    core = jax.lax.axis_index("core")
    sub = jax.lax.axis_index("subcore")
    tec_base = (core * 16 + sub) * rows_per_tec

    def in_copies(c):
        b = c % _NBUF
        rows = pl.ds(tec_base + c * _CHROWS, _CHROWS)
        return (
            pltpu.make_async_copy(gu_hbm.at[rows, :], ub.at[b], su.at[b]),
            pltpu.make_async_copy(gi_hbm.at[rows, :], vb.at[b], sv.at[b]),
        )

    def x_copy(c):
        b = c % _NBUF
        rows = pl.ds(tec_base + c * _CHROWS, _CHROWS)
        return pltpu.make_async_copy(xs.at[b], xui_hbm.at[rows], sx.at[b])

    for c in range(_PD):
        for cp in in_copies(c):
            cp.start()

    waited = set()
    for c in range(nc):
        b = c % _NBUF
        for cp in in_copies(c):
            cp.wait()

        lanes = jax.lax.broadcasted_iota(jnp.int32, (_LANES,), 0)

        @pl.loop(0, _CHROWS)
        def _(r):
            acc = ub[b, r, pl.ds(0, _LANES)] * vb[b, r, pl.ds(0, _LANES)]
            for k in range(1, D // _LANES):
                acc += (ub[b, r, pl.ds(k * _LANES, _LANES)]
                        * vb[b, r, pl.ds(k * _LANES, _LANES)])
            xacc[r, :] = acc

        @pl.loop(0, _CHROWS // _LANES)
        def _(g):
            res = jnp.zeros((_LANES,), jnp.float32)
            for l in range(_LANES):
                res = jnp.where(lanes == l,
                                jnp.sum(xacc[g * _LANES + l, :]), res)
            xs[b, pl.ds(g * _LANES, _LANES)] = res

        x_copy(c).start()
        cn = c + _PD
        if cn < nc:
            cprev = cn - _NBUF
            if cprev >= 0:
                x_copy(cprev).wait()
                waited.add(cprev)
            for cp in in_copies(cn):
                cp.start()
    for c in range(nc):
        if c not in waited:
            x_copy(c).wait()


def kernel(gu, gi):
    B, D = gu.shape
    mesh = plsc.VectorSubcoreMesh(
        core_axis_name="core", subcore_axis_name="subcore")
    cp = pltpu.CompilerParams(use_tc_tiling_on_sc=False)
    if "needs_layout_passes" in pltpu.CompilerParams.__dataclass_fields__:
        cp = dataclasses.replace(cp, needs_layout_passes=False)
    sc_kernel = pl.kernel(
        _sc_body,
        out_type=jax.ShapeDtypeStruct((B,), jnp.float32),
        mesh=mesh,
        compiler_params=cp,
        scratch_types=[
            pltpu.VMEM((_NBUF, _CHROWS, D), jnp.float32),
            pltpu.VMEM((_NBUF, _CHROWS, D), jnp.float32),
            pltpu.VMEM((_NBUF, _CHROWS), jnp.float32),
            pltpu.VMEM((_CHROWS, _LANES), jnp.float32),
            pltpu.SemaphoreType.DMA((_NBUF,)),
            pltpu.SemaphoreType.DMA((_NBUF,)),
            pltpu.SemaphoreType.DMA((_NBUF,)),
        ],
    )
    xui = sc_kernel(gu, gi)
    return (xui, gu, gi)


# speedup vs baseline: 1.7144x; 1.7144x over previous
import jax
import jax.numpy as jnp
from jax.experimental import pallas as pl


def _rowdot_kernel(gu_ref, gi_ref, out_ref):
    out_ref[:] = jnp.sum(gu_ref[:] * gi_ref[:], axis=1)


def kernel(gu, gi):
    B, D = gu.shape
    BLK = 2048
    xui = pl.pallas_call(
        _rowdot_kernel,
        grid=(B // BLK,),
        in_specs=[
            pl.BlockSpec((BLK, D), lambda i: (i, 0)),
            pl.BlockSpec((BLK, D), lambda i: (i, 0)),
        ],
        out_specs=pl.BlockSpec((BLK,), lambda i: (i,)),
        out_shape=jax.ShapeDtypeStruct((B,), jnp.float32),
    )(gu, gi)
    return (xui, gu, gi)
